# baseline (speedup 1.0000x reference)
import jax
import jax.numpy as jnp
from jax import lax
from jax.experimental import pallas as pl
from jax.experimental.pallas import tpu as pltpu
from jax.experimental.pallas import tpu_sc as plsc

F32 = jnp.float32
I32 = jnp.int32

_T1SQ = float((224.0 * 2.0 ** (-1 - 1e-6)) ** 2)
_T2SQ = float((224.0 * 2.0 ** (-1e-6)) ** 2)
_T3SQ = float((224.0 * 2.0 ** (1 - 1e-6)) ** 2)

_SIZES = (256, 128, 64, 32)
_STARTS = (0, 131072, 163840, 172032)
_SCALES = (0.25, 0.125, 0.0625, 0.03125)
_NROI = 512
_C = 256
_NCH = _C // 16


def _dyn_gather(v, idx):
    dnums = lax.GatherDimensionNumbers(
        offset_dims=(), collapsed_slice_dims=(0,), start_index_map=(0,))
    return lax.gather(v, idx[:, None], dnums, slice_sizes=(1,),
                      mode=lax.GatherScatterMode.PROMISE_IN_BOUNDS)


def _splat(v, i):
    return _dyn_gather(v, jnp.full((16,), i, I32))


def _bf16_dup(w):
    b = plsc.bitcast(w, I32)
    rne = b + jnp.full((16,), 0x7FFF, I32) + ((b >> 16) & 1)
    hi = rne & jnp.full((16,), -65536, I32)
    packed = hi | ((rne >> 16) & jnp.full((16,), 0xFFFF, I32))
    return plsc.bitcast(packed, jnp.bfloat16)


def _bf16_split(b):
    ev = lax.bitcast_convert_type(b << 16, F32)
    od = lax.bitcast_convert_type(b & jnp.full((16,), -65536, I32), F32)
    return ev, od


def _select4(sel, vals, dtype):
    out = jnp.full((16,), vals[3], dtype)
    for k in (2, 1, 0):
        out = jnp.where(sel == k, jnp.full((16,), vals[k], dtype), out)
    return out


def _side(start, binsz, wvec_i, wvec_f, off):
    lane = lax.iota(I32, 16)
    v = start + binsz * off
    valid = (v >= -1.0) & (v <= wvec_f) & (lane < 14)
    c = jnp.maximum(v, 0.0)
    lo0 = c.astype(I32)
    cond = lo0 >= wvec_i - 1
    lo = jnp.where(cond, wvec_i - 1, lo0)
    hi = jnp.where(cond, wvec_i - 1, lo0 + 1)
    cf = jnp.where(cond, wvec_f - 1.0, c)
    l = cf - lo.astype(F32)
    h = 1.0 - l
    vf = jnp.where(valid, F32(0.5), F32(0.0))
    return lo, hi, h * vf, l * vf


def _sc_body(table, boxes, out, cvm, pf, pi, idxbuf, wbuf, rbuf_a, rbuf_b,
             acc, sem_a, sem_b):
    info = plsc.get_sparse_core_info()
    nc = info.num_cores
    wid = lax.axis_index("s") * nc + lax.axis_index("c")

    pltpu.sync_copy(boxes.at[wid], cvm)
    x1 = cvm[0]
    y1 = cvm[1]
    x2 = cvm[2]
    y2 = cvm[3]

    area = (x2 - x1) * (y2 - y1)
    one = jnp.full((16,), 1, I32)
    zer = jnp.full((16,), 0, I32)
    lvl = (jnp.where(area >= _T1SQ, one, zer) + jnp.where(area >= _T2SQ, one, zer)
           + jnp.where(area >= _T3SQ, one, zer))
    scale = _select4(lvl, _SCALES, F32)
    wdim = _select4(lvl, _SIZES, I32)
    gid = wid * 16 + lax.iota(I32, 16)
    x1s = x1 * scale
    y1s = y1 * scale
    pf[0] = x1s
    pf[1] = y1s
    pf[2] = jnp.maximum(x2 * scale - x1s, 1.0) / 7.0
    pf[3] = jnp.maximum(y2 * scale - y1s, 1.0) / 7.0
    pf[4] = _select4(lvl, [float(v) for v in _SIZES], F32)
    pi[0] = wdim
    pi[1] = _select4(lvl, _STARTS, I32) + jnp.where(gid >= 256, wdim * wdim, zer)

    def roi_body(r, carry):
        off = lax.iota(I32, 16).astype(F32) * 0.5 + 0.25
        wdim_b = _splat(pi[0], r)
        wdim_fb = _splat(pf[4], r)
        base_b = _splat(pi[1], r)
        xlo, xhi, wxl, wxh = _side(_splat(pf[0], r), _splat(pf[2], r), wdim_b,
                                   wdim_fb, off)
        ylo, yhi, wyl, wyh = _side(_splat(pf[1], r), _splat(pf[3], r), wdim_b,
                                   wdim_fb, off)

        for jy in range(15):
            rowlo = base_b + _splat(ylo, jy) * wdim_b
            rowhi = base_b + _splat(yhi, jy) * wdim_b
            idxbuf[jy, pl.ds(0, 16)] = rowlo + xlo
            idxbuf[jy, pl.ds(16, 16)] = rowlo + xhi
            idxbuf[jy, pl.ds(32, 16)] = rowhi + xlo
            idxbuf[jy, pl.ds(48, 16)] = rowhi + xhi
            if jy < 14:
                wyl_b = _splat(wyl, jy)
                wyh_b = _splat(wyh, jy)
                wbuf[4 * jy + 0] = wyl_b * wxl
                wbuf[4 * jy + 1] = wyl_b * wxh
                wbuf[4 * jy + 2] = wyh_b * wxl
                wbuf[4 * jy + 3] = wyh_b * wxh

        def fire(jy, buf, s):
            pltpu.async_copy(table.at[idxbuf.at[jy]], buf, s)

        def compute_row(jy, byrow, buf, first):
            wc = (wbuf[4 * jy + 0], wbuf[4 * jy + 1],
                  wbuf[4 * jy + 2], wbuf[4 * jy + 3])
            for bx in range(7):
                ws = []
                rows = []
                for c in range(4):
                    for p in (2 * bx, 2 * bx + 1):
                        ws.append(_splat(wc[c], p))
                        rows.append(c * 16 + p)
                row0 = byrow * 7 + bx

                def lds(p):
                    return [buf[rr, pl.ds(16 * p, 16)] for rr in rows]

                cur = lds(0)
                for p in range(_NCH // 2):
                    nxt = lds(p + 1) if p + 1 < _NCH // 2 else None
                    sp = [_bf16_split(x) for x in cur]
                    ev = ((ws[0] * sp[0][0] + ws[1] * sp[1][0])
                          + (ws[2] * sp[2][0] + ws[3] * sp[3][0])
                          + ((ws[4] * sp[4][0] + ws[5] * sp[5][0])
                             + (ws[6] * sp[6][0] + ws[7] * sp[7][0])))
                    od = ((ws[0] * sp[0][1] + ws[1] * sp[1][1])
                          + (ws[2] * sp[2][1] + ws[3] * sp[3][1])
                          + ((ws[4] * sp[4][1] + ws[5] * sp[5][1])
                             + (ws[6] * sp[6][1] + ws[7] * sp[7][1])))
                    se = pl.ds(32 * p, 16)
                    so = pl.ds(32 * p + 16, 16)
                    if first:
                        acc[row0, se] = ev
                        acc[row0, so] = od
                    else:
                        plsc.addupdate(acc.at[row0, se], ev)
                        plsc.addupdate(acc.at[row0, so], od)
                    cur = nxt

        fire(0, rbuf_a, sem_a)

        def jj_body(jj, carry2):
            jy = 2 * jj
            fire(jy + 1, rbuf_b, sem_b)
            pltpu.make_async_copy(table.at[idxbuf.at[jy]], rbuf_a, sem_a).wait()
            compute_row(jy, jj, rbuf_a, True)
            fire(jy + 2, rbuf_a, sem_a)
            pltpu.make_async_copy(table.at[idxbuf.at[jy]], rbuf_b, sem_b).wait()
            compute_row(jy + 1, jj, rbuf_b, False)
            return carry2
        lax.fori_loop(0, 7, jj_body, 0)
        pltpu.make_async_copy(table.at[idxbuf.at[0]], rbuf_a, sem_a).wait()
        pltpu.sync_copy(acc, out.at[wid * 16 + r])
        return carry
    lax.fori_loop(0, 16, roi_body, 0)


@jax.jit
def _roipool(table, boxes):
    mesh = plsc.VectorSubcoreMesh(core_axis_name="c", subcore_axis_name="s")
    fn = pl.kernel(
        _sc_body, mesh=mesh,
        out_type=jax.ShapeDtypeStruct((_NROI, 49, _C), F32),
        scratch_types=[
            pltpu.VMEM((4, 16), F32),
            pltpu.VMEM((5, 16), F32),
            pltpu.VMEM((2, 16), I32),
            pltpu.VMEM((15, 64), I32),
            pltpu.VMEM((56, 16), F32),
            pltpu.VMEM((64, _C // 2), I32),
            pltpu.VMEM((64, _C // 2), I32),
            pltpu.VMEM((49, _C), F32),
            pltpu.SemaphoreType.DMA,
            pltpu.SemaphoreType.DMA,
        ],
    )
    return fn(table, boxes)


def kernel(features_0, features_1, features_2, features_3, boxes_0, boxes_1):
    feats = (features_0, features_1, features_2, features_3)
    table = jnp.concatenate(
        [jnp.transpose(f, (0, 2, 3, 1)).reshape(-1, _C) for f in feats],
        0).astype(jnp.bfloat16)
    table = lax.bitcast_convert_type(
        table.reshape(-1, _C // 2, 2), I32)
    boxes = (jnp.concatenate([boxes_0, boxes_1], 0)
             .reshape(32, 16, 4).transpose(0, 2, 1))
    out = _roipool(table, boxes)
    out = (out.reshape(_NROI, 49, 8, 2, 16).transpose(0, 1, 2, 4, 3)
           .reshape(_NROI, 49, _C))
    return out.transpose(0, 2, 1).reshape(_NROI, _C, 7, 7)


# speedup vs baseline: 1.9444x; 1.9444x over previous
import jax
import jax.numpy as jnp
from jax import lax
from jax.experimental import pallas as pl
from jax.experimental.pallas import tpu as pltpu
from jax.experimental.pallas import tpu_sc as plsc

F32 = jnp.float32
I32 = jnp.int32

_T1SQ = float((224.0 * 2.0 ** (-1 - 1e-6)) ** 2)
_T2SQ = float((224.0 * 2.0 ** (-1e-6)) ** 2)
_T3SQ = float((224.0 * 2.0 ** (1 - 1e-6)) ** 2)

_SIZES = (256, 128, 64, 32)
_STARTS = (0, 131072, 163840, 172032)
_SCALES = (0.25, 0.125, 0.0625, 0.03125)
_NROI = 512
_C = 256
_NCH = _C // 16


def _dyn_gather(v, idx):
    dnums = lax.GatherDimensionNumbers(
        offset_dims=(), collapsed_slice_dims=(0,), start_index_map=(0,))
    return lax.gather(v, idx[:, None], dnums, slice_sizes=(1,),
                      mode=lax.GatherScatterMode.PROMISE_IN_BOUNDS)


def _splat(v, i):
    return _dyn_gather(v, jnp.full((16,), i, I32))


def _bf16_dup(w):
    b = plsc.bitcast(w, I32)
    rne = b + jnp.full((16,), 0x7FFF, I32) + ((b >> 16) & 1)
    hi = rne & jnp.full((16,), -65536, I32)
    packed = hi | ((rne >> 16) & jnp.full((16,), 0xFFFF, I32))
    return plsc.bitcast(packed, jnp.bfloat16)


def _bf16_split(b):
    lo = lax.bitcast_convert_type(b << 16, F32)
    hi = lax.bitcast_convert_type(b & jnp.full((16,), -65536, I32), F32)
    return lo, hi


def _select4(sel, vals, dtype):
    out = jnp.full((16,), vals[3], dtype)
    for k in (2, 1, 0):
        out = jnp.where(sel == k, jnp.full((16,), vals[k], dtype), out)
    return out


def _side(start, binsz, wvec_i, wvec_f, off):
    lane = lax.iota(I32, 16)
    v = start + binsz * off
    valid = (v >= -1.0) & (v <= wvec_f) & (lane < 14)
    c = jnp.maximum(v, 0.0)
    lo0 = c.astype(I32)
    cond = lo0 >= wvec_i - 1
    lo = jnp.where(cond, wvec_i - 1, lo0)
    hi = jnp.where(cond, wvec_i - 1, lo0 + 1)
    cf = jnp.where(cond, wvec_f - 1.0, c)
    l = cf - lo.astype(F32)
    h = 1.0 - l
    vf = jnp.where(valid, F32(0.5), F32(0.0))
    return lo, hi, h * vf, l * vf


def _sc_body(table, boxes, out, cvm, pf, pi, idxbuf, wbuf, rbuf_a, rbuf_b,
             acc, sem_a, sem_b):
    info = plsc.get_sparse_core_info()
    nc = info.num_cores
    wid = lax.axis_index("s") * nc + lax.axis_index("c")

    pltpu.sync_copy(boxes.at[wid], cvm)
    x1 = cvm[0]
    y1 = cvm[1]
    x2 = cvm[2]
    y2 = cvm[3]

    area = (x2 - x1) * (y2 - y1)
    one = jnp.full((16,), 1, I32)
    zer = jnp.full((16,), 0, I32)
    lvl = (jnp.where(area >= _T1SQ, one, zer) + jnp.where(area >= _T2SQ, one, zer)
           + jnp.where(area >= _T3SQ, one, zer))
    scale = _select4(lvl, _SCALES, F32)
    wdim = _select4(lvl, _SIZES, I32)
    gid = wid * 16 + lax.iota(I32, 16)
    x1s = x1 * scale
    y1s = y1 * scale
    pf[0] = x1s
    pf[1] = y1s
    pf[2] = jnp.maximum(x2 * scale - x1s, 1.0) / 7.0
    pf[3] = jnp.maximum(y2 * scale - y1s, 1.0) / 7.0
    pf[4] = _select4(lvl, [float(v) for v in _SIZES], F32)
    pi[0] = wdim
    pi[1] = _select4(lvl, _STARTS, I32) + jnp.where(gid >= 256, wdim * wdim, zer)

    def roi_body(r, carry):
        off = lax.iota(I32, 16).astype(F32) * 0.5 + 0.25
        wdim_b = _splat(pi[0], r)
        wdim_fb = _splat(pf[4], r)
        base_b = _splat(pi[1], r)
        xlo, xhi, wxl, wxh = _side(_splat(pf[0], r), _splat(pf[2], r), wdim_b,
                                   wdim_fb, off)
        ylo, yhi, wyl, wyh = _side(_splat(pf[1], r), _splat(pf[3], r), wdim_b,
                                   wdim_fb, off)

        for jy in range(15):
            rowlo = base_b + _splat(ylo, jy) * wdim_b
            rowhi = base_b + _splat(yhi, jy) * wdim_b
            idxbuf[jy, pl.ds(0, 16)] = rowlo + xlo
            idxbuf[jy, pl.ds(16, 16)] = rowlo + xhi
            idxbuf[jy, pl.ds(32, 16)] = rowhi + xlo
            idxbuf[jy, pl.ds(48, 16)] = rowhi + xhi
            if jy < 14:
                wyl_b = _splat(wyl, jy)
                wyh_b = _splat(wyh, jy)
                wbuf[4 * jy + 0] = wyl_b * wxl
                wbuf[4 * jy + 1] = wyl_b * wxh
                wbuf[4 * jy + 2] = wyh_b * wxl
                wbuf[4 * jy + 3] = wyh_b * wxh

        def fire(jy, buf, s):
            pltpu.async_copy(table.at[idxbuf.at[jy]], buf, s)

        def compute_row(jy, byrow, buf, first):
            wc = (wbuf[4 * jy + 0], wbuf[4 * jy + 1],
                  wbuf[4 * jy + 2], wbuf[4 * jy + 3])
            for bx in range(7):
                ws = []
                rows = []
                for c in range(4):
                    for p in (2 * bx, 2 * bx + 1):
                        ws.append(_splat(wc[c], p))
                        rows.append(c * 16 + p)
                row0 = byrow * 7 + bx

                def lds(p):
                    return [buf[rr, pl.ds(16 * p, 16)] for rr in rows]

                cur = lds(0)
                for p in range(_NCH // 2):
                    nxt = lds(p + 1) if p + 1 < _NCH // 2 else None
                    sp = [_bf16_split(x) for x in cur]
                    lo = ((ws[0] * sp[0][0] + ws[1] * sp[1][0])
                          + (ws[2] * sp[2][0] + ws[3] * sp[3][0])
                          + ((ws[4] * sp[4][0] + ws[5] * sp[5][0])
                             + (ws[6] * sp[6][0] + ws[7] * sp[7][0])))
                    hi = ((ws[0] * sp[0][1] + ws[1] * sp[1][1])
                          + (ws[2] * sp[2][1] + ws[3] * sp[3][1])
                          + ((ws[4] * sp[4][1] + ws[5] * sp[5][1])
                             + (ws[6] * sp[6][1] + ws[7] * sp[7][1])))
                    sl = pl.ds(16 * p, 16)
                    sh = pl.ds(128 + 16 * p, 16)
                    if first:
                        acc[row0, sl] = lo
                        acc[row0, sh] = hi
                    else:
                        plsc.addupdate(acc.at[row0, sl], lo)
                        plsc.addupdate(acc.at[row0, sh], hi)
                    cur = nxt

        fire(0, rbuf_a, sem_a)

        def jj_body(jj, carry2):
            jy = 2 * jj
            fire(jy + 1, rbuf_b, sem_b)
            pltpu.make_async_copy(table.at[idxbuf.at[jy]], rbuf_a, sem_a).wait()
            compute_row(jy, jj, rbuf_a, True)
            fire(jy + 2, rbuf_a, sem_a)
            pltpu.make_async_copy(table.at[idxbuf.at[jy]], rbuf_b, sem_b).wait()
            compute_row(jy + 1, jj, rbuf_b, False)
            return carry2
        lax.fori_loop(0, 7, jj_body, 0)
        pltpu.make_async_copy(table.at[idxbuf.at[0]], rbuf_a, sem_a).wait()
        pltpu.sync_copy(acc, out.at[wid * 16 + r])
        return carry
    lax.fori_loop(0, 16, roi_body, 0)


@jax.jit
def _roipool(table, boxes):
    mesh = plsc.VectorSubcoreMesh(core_axis_name="c", subcore_axis_name="s")
    fn = pl.kernel(
        _sc_body, mesh=mesh,
        out_type=jax.ShapeDtypeStruct((_NROI, 49, _C), F32),
        scratch_types=[
            pltpu.VMEM((4, 16), F32),
            pltpu.VMEM((5, 16), F32),
            pltpu.VMEM((2, 16), I32),
            pltpu.VMEM((15, 64), I32),
            pltpu.VMEM((56, 16), F32),
            pltpu.VMEM((64, _C // 2), I32),
            pltpu.VMEM((64, _C // 2), I32),
            pltpu.VMEM((49, _C), F32),
            pltpu.SemaphoreType.DMA,
            pltpu.SemaphoreType.DMA,
        ],
    )
    return fn(table, boxes)


def kernel(features_0, features_1, features_2, features_3, boxes_0, boxes_1):
    feats = (features_0, features_1, features_2, features_3)
    table = jnp.concatenate(
        [jnp.transpose(f, (0, 2, 3, 1)).reshape(-1, _C) for f in feats], 0)
    b = lax.bitcast_convert_type(table, I32)
    rne = b + 0x7FFF + ((b >> 16) & 1)
    code = (rne >> 16) & 0xFFFF
    table = code[:, : _C // 2] | (code[:, _C // 2:] << 16)
    boxes = (jnp.concatenate([boxes_0, boxes_1], 0)
             .reshape(32, 16, 4).transpose(0, 2, 1))
    out = _roipool(table, boxes)
    return out.transpose(0, 2, 1).reshape(_NROI, _C, 7, 7)


# speedup vs baseline: 2.0291x; 1.0436x over previous
import jax
import jax.numpy as jnp
from jax import lax
from jax.experimental import pallas as pl
from jax.experimental.pallas import tpu as pltpu
from jax.experimental.pallas import tpu_sc as plsc

F32 = jnp.float32
I32 = jnp.int32

_T1SQ = float((224.0 * 2.0 ** (-1 - 1e-6)) ** 2)
_T2SQ = float((224.0 * 2.0 ** (-1e-6)) ** 2)
_T3SQ = float((224.0 * 2.0 ** (1 - 1e-6)) ** 2)

_SIZES = (256, 128, 64, 32)
_STARTS = (0, 131072, 163840, 172032)
_SCALES = (0.25, 0.125, 0.0625, 0.03125)
_NROI = 512
_C = 256
_NCH = _C // 16


def _dyn_gather(v, idx):
    dnums = lax.GatherDimensionNumbers(
        offset_dims=(), collapsed_slice_dims=(0,), start_index_map=(0,))
    return lax.gather(v, idx[:, None], dnums, slice_sizes=(1,),
                      mode=lax.GatherScatterMode.PROMISE_IN_BOUNDS)


def _splat(v, i):
    return _dyn_gather(v, jnp.full((16,), i, I32))


def _select4(sel, vals, dtype):
    out = jnp.full((16,), vals[3], dtype)
    for k in (2, 1, 0):
        out = jnp.where(sel == k, jnp.full((16,), vals[k], dtype), out)
    return out


def _side(start, binsz, wvec_i, wvec_f, off):
    lane = lax.iota(I32, 16)
    v = start + binsz * off
    valid = (v >= -1.0) & (v <= wvec_f) & (lane < 14)
    c = jnp.maximum(v, 0.0)
    lo0 = c.astype(I32)
    cond = lo0 >= wvec_i - 1
    lo = jnp.where(cond, wvec_i - 1, lo0)
    hi = jnp.where(cond, wvec_i - 1, lo0 + 1)
    cf = jnp.where(cond, wvec_f - 1.0, c)
    l = cf - lo.astype(F32)
    h = 1.0 - l
    vf = jnp.where(valid, F32(0.5), F32(0.0))
    return lo, hi, h * vf, l * vf


def _sc_body(table, boxes, out, cvm, pf, pi, idxbuf, wbuf, rbuf_a, rbuf_b,
             acc, sem_a, sem_b):
    info = plsc.get_sparse_core_info()
    nc = info.num_cores
    wid = lax.axis_index("s") * nc + lax.axis_index("c")

    pltpu.sync_copy(boxes.at[wid], cvm)
    x1 = cvm[0]
    y1 = cvm[1]
    x2 = cvm[2]
    y2 = cvm[3]

    area = (x2 - x1) * (y2 - y1)
    one = jnp.full((16,), 1, I32)
    zer = jnp.full((16,), 0, I32)
    lvl = (jnp.where(area >= _T1SQ, one, zer) + jnp.where(area >= _T2SQ, one, zer)
           + jnp.where(area >= _T3SQ, one, zer))
    scale = _select4(lvl, _SCALES, F32)
    wdim = _select4(lvl, _SIZES, I32)
    gid = wid * 16 + lax.iota(I32, 16)
    x1s = x1 * scale
    y1s = y1 * scale
    pf[0] = x1s
    pf[1] = y1s
    pf[2] = jnp.maximum(x2 * scale - x1s, 1.0) / 7.0
    pf[3] = jnp.maximum(y2 * scale - y1s, 1.0) / 7.0
    pf[4] = _select4(lvl, [float(v) for v in _SIZES], F32)
    pi[0] = wdim
    pi[1] = _select4(lvl, _STARTS, I32) + jnp.where(gid >= 256, wdim * wdim, zer)

    def roi_body(r, carry):
        off = lax.iota(I32, 16).astype(F32) * 0.5 + 0.25
        wdim_b = _splat(pi[0], r)
        wdim_fb = _splat(pf[4], r)
        base_b = _splat(pi[1], r)
        xlo, xhi, wxl, wxh = _side(_splat(pf[0], r), _splat(pf[2], r), wdim_b,
                                   wdim_fb, off)
        ylo, yhi, wyl, wyh = _side(_splat(pf[1], r), _splat(pf[3], r), wdim_b,
                                   wdim_fb, off)

        for jy in range(15):
            rowlo = base_b + _splat(ylo, jy) * wdim_b
            rowhi = base_b + _splat(yhi, jy) * wdim_b
            idxbuf[jy, pl.ds(0, 16)] = rowlo + xlo
            idxbuf[jy, pl.ds(16, 16)] = rowlo + xhi
            idxbuf[jy, pl.ds(32, 16)] = rowhi + xlo
            idxbuf[jy, pl.ds(48, 16)] = rowhi + xhi
            if jy < 14:
                wyl_b = _splat(wyl, jy)
                wyh_b = _splat(wyh, jy)
                wbuf[4 * jy + 0] = wyl_b * wxl
                wbuf[4 * jy + 1] = wyl_b * wxh
                wbuf[4 * jy + 2] = wyh_b * wxl
                wbuf[4 * jy + 3] = wyh_b * wxh

        def fire(jy, buf, s):
            pltpu.async_copy(table.at[idxbuf.at[jy]], buf, s)

        def compute_row(jy, byrow, buf, first):
            wc = (wbuf[4 * jy + 0], wbuf[4 * jy + 1],
                  wbuf[4 * jy + 2], wbuf[4 * jy + 3])
            for bx in range(7):
                ws = []
                rows = []
                for c in range(4):
                    for p in (2 * bx, 2 * bx + 1):
                        ws.append(_splat(wc[c], p))
                        rows.append(c * 16 + p)
                row0 = byrow * 7 + bx

                def lds(p):
                    return [[buf[rr, pl.ds((2 * p + h) * 16, 16)]
                             for rr in rows] for h in (0, 1)]

                cur = lds(0)
                for p in range(_NCH // 2):
                    nxt = lds(p + 1) if p + 1 < _NCH // 2 else None
                    for h in (0, 1):
                        L = cur[h]
                        t0 = ws[0] * L[0] + ws[1] * L[1]
                        t1 = ws[2] * L[2] + ws[3] * L[3]
                        t2 = ws[4] * L[4] + ws[5] * L[5]
                        t3 = ws[6] * L[6] + ws[7] * L[7]
                        contrib = (t0 + t1) + (t2 + t3)
                        sl = pl.ds((2 * p + h) * 16, 16)
                        if first:
                            acc[row0, sl] = contrib
                        else:
                            plsc.addupdate(acc.at[row0, sl], contrib)
                    cur = nxt

        fire(0, rbuf_a, sem_a)

        def jj_body(jj, carry2):
            jy = 2 * jj
            fire(jy + 1, rbuf_b, sem_b)
            pltpu.make_async_copy(table.at[idxbuf.at[jy]], rbuf_a, sem_a).wait()
            compute_row(jy, jj, rbuf_a, True)
            fire(jy + 2, rbuf_a, sem_a)
            pltpu.make_async_copy(table.at[idxbuf.at[jy]], rbuf_b, sem_b).wait()
            compute_row(jy + 1, jj, rbuf_b, False)
            return carry2
        lax.fori_loop(0, 7, jj_body, 0)
        pltpu.make_async_copy(table.at[idxbuf.at[0]], rbuf_a, sem_a).wait()
        pltpu.sync_copy(acc, out.at[wid * 16 + r])
        return carry
    lax.fori_loop(0, 16, roi_body, 0)


@jax.jit
def _roipool(table, boxes):
    mesh = plsc.VectorSubcoreMesh(core_axis_name="c", subcore_axis_name="s")
    fn = pl.kernel(
        _sc_body, mesh=mesh,
        out_type=jax.ShapeDtypeStruct((_NROI, 49, _C), F32),
        scratch_types=[
            pltpu.VMEM((4, 16), F32),
            pltpu.VMEM((5, 16), F32),
            pltpu.VMEM((2, 16), I32),
            pltpu.VMEM((15, 64), I32),
            pltpu.VMEM((56, 16), F32),
            pltpu.VMEM((64, _C), F32),
            pltpu.VMEM((64, _C), F32),
            pltpu.VMEM((49, _C), F32),
            pltpu.SemaphoreType.DMA,
            pltpu.SemaphoreType.DMA,
        ],
    )
    return fn(table, boxes)


def kernel(features_0, features_1, features_2, features_3, boxes_0, boxes_1):
    feats = (features_0, features_1, features_2, features_3)
    table = jnp.concatenate(
        [jnp.transpose(f, (0, 2, 3, 1)).reshape(-1, _C) for f in feats], 0)
    boxes = (jnp.concatenate([boxes_0, boxes_1], 0)
             .reshape(32, 16, 4).transpose(0, 2, 1))
    out = _roipool(table, boxes)
    return out.transpose(0, 2, 1).reshape(_NROI, _C, 7, 7)


# speedup vs baseline: 2.1397x; 1.0545x over previous
import jax
import jax.numpy as jnp
from jax import lax
from jax.experimental import pallas as pl
from jax.experimental.pallas import tpu as pltpu
from jax.experimental.pallas import tpu_sc as plsc

F32 = jnp.float32
I32 = jnp.int32

_T1SQ = float((224.0 * 2.0 ** (-1 - 1e-6)) ** 2)
_T2SQ = float((224.0 * 2.0 ** (-1e-6)) ** 2)
_T3SQ = float((224.0 * 2.0 ** (1 - 1e-6)) ** 2)

_SIZES = (256, 128, 64, 32)
_STARTS = (0, 131072, 163840, 172032)
_SCALES = (0.25, 0.125, 0.0625, 0.03125)
_NROI = 512
_C = 256
_NCH = _C // 16


def _dyn_gather(v, idx):
    dnums = lax.GatherDimensionNumbers(
        offset_dims=(), collapsed_slice_dims=(0,), start_index_map=(0,))
    return lax.gather(v, idx[:, None], dnums, slice_sizes=(1,),
                      mode=lax.GatherScatterMode.PROMISE_IN_BOUNDS)


def _splat(v, i):
    return _dyn_gather(v, jnp.full((16,), i, I32))


def _select4(sel, vals, dtype):
    out = jnp.full((16,), vals[3], dtype)
    for k in (2, 1, 0):
        out = jnp.where(sel == k, jnp.full((16,), vals[k], dtype), out)
    return out


def _side(start, binsz, wvec_i, wvec_f, off):
    lane = lax.iota(I32, 16)
    v = start + binsz * off
    valid = (v >= -1.0) & (v <= wvec_f) & (lane < 14)
    c = jnp.maximum(v, 0.0)
    lo0 = c.astype(I32)
    cond = lo0 >= wvec_i - 1
    lo = jnp.where(cond, wvec_i - 1, lo0)
    hi = jnp.where(cond, wvec_i - 1, lo0 + 1)
    cf = jnp.where(cond, wvec_f - 1.0, c)
    l = cf - lo.astype(F32)
    h = 1.0 - l
    vf = jnp.where(valid, F32(0.5), F32(0.0))
    return lo, hi, h * vf, l * vf


def _sc_body(table, boxes, out, cvm, pf, pi, idxbuf, wbuf, rbuf_a, rbuf_b,
             acc, sem_a, sem_b):
    info = plsc.get_sparse_core_info()
    nc = info.num_cores
    wid = lax.axis_index("s") * nc + lax.axis_index("c")

    pltpu.sync_copy(boxes.at[wid], cvm)
    x1 = cvm[0]
    y1 = cvm[1]
    x2 = cvm[2]
    y2 = cvm[3]

    area = (x2 - x1) * (y2 - y1)
    one = jnp.full((16,), 1, I32)
    zer = jnp.full((16,), 0, I32)
    lvl = (jnp.where(area >= _T1SQ, one, zer) + jnp.where(area >= _T2SQ, one, zer)
           + jnp.where(area >= _T3SQ, one, zer))
    scale = _select4(lvl, _SCALES, F32)
    wdim = _select4(lvl, _SIZES, I32)
    gid = wid * 16 + lax.iota(I32, 16)
    x1s = x1 * scale
    y1s = y1 * scale
    pf[0] = x1s
    pf[1] = y1s
    pf[2] = jnp.maximum(x2 * scale - x1s, 1.0) / 7.0
    pf[3] = jnp.maximum(y2 * scale - y1s, 1.0) / 7.0
    pf[4] = _select4(lvl, [float(v) for v in _SIZES], F32)
    pi[0] = wdim
    pi[1] = _select4(lvl, _STARTS, I32) + jnp.where(gid >= 256, wdim * wdim, zer)

    def roi_body(r, carry):
        off = lax.iota(I32, 16).astype(F32) * 0.5 + 0.25
        wdim_b = _splat(pi[0], r)
        wdim_fb = _splat(pf[4], r)
        base_b = _splat(pi[1], r)
        xlo, xhi, wxl, wxh = _side(_splat(pf[0], r), _splat(pf[2], r), wdim_b,
                                   wdim_fb, off)
        ylo, yhi, wyl, wyh = _side(_splat(pf[1], r), _splat(pf[3], r), wdim_b,
                                   wdim_fb, off)

        for jy in range(15):
            rowlo = base_b + _splat(ylo, jy) * wdim_b
            rowhi = base_b + _splat(yhi, jy) * wdim_b
            idxbuf[jy, pl.ds(0, 16)] = rowlo + xlo
            idxbuf[jy, pl.ds(16, 16)] = rowlo + xhi
            idxbuf[jy, pl.ds(32, 16)] = rowhi + xlo
            idxbuf[jy, pl.ds(48, 16)] = rowhi + xhi
            if jy < 14:
                wyl_b = _splat(wyl, jy)
                wyh_b = _splat(wyh, jy)
                wbuf[4 * jy + 0] = wyl_b * wxl
                wbuf[4 * jy + 1] = wyl_b * wxh
                wbuf[4 * jy + 2] = wyh_b * wxl
                wbuf[4 * jy + 3] = wyh_b * wxh

        def fire(jy, buf, s):
            pltpu.async_copy(table.at[idxbuf.at[jy]], buf, s)

        def compute_row(jy, byrow, buf, first):
            wc = (wbuf[4 * jy + 0], wbuf[4 * jy + 1],
                  wbuf[4 * jy + 2], wbuf[4 * jy + 3])
            for bx in range(7):
                ws = []
                rows = []
                for c in range(4):
                    for p in (2 * bx, 2 * bx + 1):
                        ws.append(_splat(wc[c], p))
                        rows.append(c * 16 + p)
                row0 = byrow * 7 + bx

                def lds(p):
                    return [[buf[rr, pl.ds((2 * p + h) * 16, 16)]
                             for rr in rows] for h in (0, 1)]

                cur = lds(0)
                for p in range(_NCH // 2):
                    nxt = lds(p + 1) if p + 1 < _NCH // 2 else None
                    for h in (0, 1):
                        L = cur[h]
                        t0 = ws[0] * L[0] + ws[1] * L[1]
                        t1 = ws[2] * L[2] + ws[3] * L[3]
                        t2 = ws[4] * L[4] + ws[5] * L[5]
                        t3 = ws[6] * L[6] + ws[7] * L[7]
                        contrib = (t0 + t1) + (t2 + t3)
                        sl = pl.ds((2 * p + h) * 16, 16)
                        if first:
                            acc[row0, sl] = contrib
                        else:
                            plsc.addupdate(acc.at[row0, sl], contrib)
                    cur = nxt

        fire(0, rbuf_a, sem_a)

        def jj_body(jj, carry2):
            jy = 2 * jj
            fire(jy + 1, rbuf_b, sem_b)
            pltpu.make_async_copy(table.at[idxbuf.at[jy]], rbuf_a, sem_a).wait()
            compute_row(jy, jj, rbuf_a, True)
            fire(jy + 2, rbuf_a, sem_a)
            pltpu.make_async_copy(table.at[idxbuf.at[jy]], rbuf_b, sem_b).wait()
            compute_row(jy + 1, jj, rbuf_b, False)
            return carry2
        lax.fori_loop(0, 7, jj_body, 0)
        pltpu.make_async_copy(table.at[idxbuf.at[0]], rbuf_a, sem_a).wait()
        pltpu.sync_copy(acc, out.at[wid * 16 + r])
        return carry
    lax.fori_loop(0, 16, roi_body, 0)


def _tc_pack_level(feat, table, start_row, yb):
    B, C, S, _ = feat.shape
    rows_per_step = yb * S
    base = start_row // rows_per_step

    def body(f_ref, t_in_ref, t_out_ref):
        x = f_ref[0].reshape(C, yb * S)
        t_out_ref[...] = jnp.transpose(x, (1, 0))

    grid = (B, S // yb)
    return pl.pallas_call(
        body,
        grid=grid,
        in_specs=[
            pl.BlockSpec((1, C, yb, S), lambda b, g: (b, 0, g, 0)),
            pl.BlockSpec(memory_space=pl.ANY),
        ],
        out_specs=pl.BlockSpec((rows_per_step, C),
                               lambda b, g, _S=S, _yb=yb, _base=base:
                               (_base + b * (_S // _yb) + g, 0)),
        out_shape=jax.ShapeDtypeStruct(table.shape, table.dtype),
        input_output_aliases={1: 0},
    )(feat, table)


@jax.jit
def _roipool(table, boxes):
    mesh = plsc.VectorSubcoreMesh(core_axis_name="c", subcore_axis_name="s")
    fn = pl.kernel(
        _sc_body, mesh=mesh,
        out_type=jax.ShapeDtypeStruct((_NROI, 49, _C), F32),
        scratch_types=[
            pltpu.VMEM((4, 16), F32),
            pltpu.VMEM((5, 16), F32),
            pltpu.VMEM((2, 16), I32),
            pltpu.VMEM((15, 64), I32),
            pltpu.VMEM((56, 16), F32),
            pltpu.VMEM((64, _C), F32),
            pltpu.VMEM((64, _C), F32),
            pltpu.VMEM((49, _C), F32),
            pltpu.SemaphoreType.DMA,
            pltpu.SemaphoreType.DMA,
        ],
    )
    return fn(table, boxes)


def kernel(features_0, features_1, features_2, features_3, boxes_0, boxes_1):
    feats = (features_0, features_1, features_2, features_3)
    table = jnp.zeros((174080, _C), F32)
    for f, st, yb in zip(feats, _STARTS, (8, 16, 32, 32)):
        table = _tc_pack_level(f, table, st, yb)
    boxes = (jnp.concatenate([boxes_0, boxes_1], 0)
             .reshape(32, 16, 4).transpose(0, 2, 1))
    out = _roipool(table, boxes)
    return out.transpose(0, 2, 1).reshape(_NROI, _C, 7, 7)


# speedup vs baseline: 2.3060x; 1.0777x over previous
import jax
import jax.numpy as jnp
from jax import lax
from jax.experimental import pallas as pl
from jax.experimental.pallas import tpu as pltpu
from jax.experimental.pallas import tpu_sc as plsc

F32 = jnp.float32
I32 = jnp.int32

_T1SQ = float((224.0 * 2.0 ** (-1 - 1e-6)) ** 2)
_T2SQ = float((224.0 * 2.0 ** (-1e-6)) ** 2)
_T3SQ = float((224.0 * 2.0 ** (1 - 1e-6)) ** 2)

_SIZES = (256, 128, 64, 32)
_STARTS = (0, 131072, 163840, 172032)
_SCALES = (0.25, 0.125, 0.0625, 0.03125)
_NROI = 512
_C = 256
_NCH = _C // 16


def _dyn_gather(v, idx):
    dnums = lax.GatherDimensionNumbers(
        offset_dims=(), collapsed_slice_dims=(0,), start_index_map=(0,))
    return lax.gather(v, idx[:, None], dnums, slice_sizes=(1,),
                      mode=lax.GatherScatterMode.PROMISE_IN_BOUNDS)


def _splat(v, i):
    return _dyn_gather(v, jnp.full((16,), i, I32))


def _select4(sel, vals, dtype):
    out = jnp.full((16,), vals[3], dtype)
    for k in (2, 1, 0):
        out = jnp.where(sel == k, jnp.full((16,), vals[k], dtype), out)
    return out


def _side(start, binsz, wvec_i, wvec_f, off):
    lane = lax.iota(I32, 16)
    v = start + binsz * off
    valid = (v >= -1.0) & (v <= wvec_f) & (lane < 14)
    c = jnp.maximum(v, 0.0)
    lo0 = c.astype(I32)
    cond = lo0 >= wvec_i - 1
    lo = jnp.where(cond, wvec_i - 1, lo0)
    hi = jnp.where(cond, wvec_i - 1, lo0 + 1)
    cf = jnp.where(cond, wvec_f - 1.0, c)
    l = cf - lo.astype(F32)
    h = 1.0 - l
    vf = jnp.where(valid, F32(0.5), F32(0.0))
    return lo, hi, h * vf, l * vf


def _sc_body(table, boxes, out, cvm, pf, pi, idxbuf, wbuf, rbuf_a, rbuf_b,
             acc, sem_a, sem_b):
    info = plsc.get_sparse_core_info()
    nc = info.num_cores
    wid = lax.axis_index("s") * nc + lax.axis_index("c")

    pltpu.sync_copy(boxes.at[wid], cvm)
    x1 = cvm[0]
    y1 = cvm[1]
    x2 = cvm[2]
    y2 = cvm[3]

    area = (x2 - x1) * (y2 - y1)
    one = jnp.full((16,), 1, I32)
    zer = jnp.full((16,), 0, I32)
    lvl = (jnp.where(area >= _T1SQ, one, zer) + jnp.where(area >= _T2SQ, one, zer)
           + jnp.where(area >= _T3SQ, one, zer))
    scale = _select4(lvl, _SCALES, F32)
    wdim = _select4(lvl, _SIZES, I32)
    gid = wid * 16 + lax.iota(I32, 16)
    x1s = x1 * scale
    y1s = y1 * scale
    pf[0] = x1s
    pf[1] = y1s
    pf[2] = jnp.maximum(x2 * scale - x1s, 1.0) / 7.0
    pf[3] = jnp.maximum(y2 * scale - y1s, 1.0) / 7.0
    pf[4] = _select4(lvl, [float(v) for v in _SIZES], F32)
    pi[0] = wdim
    pi[1] = _select4(lvl, _STARTS, I32) + jnp.where(gid >= 256, wdim * wdim, zer)

    def roi_body(r, carry):
        off = lax.iota(I32, 16).astype(F32) * 0.5 + 0.25
        wdim_b = _splat(pi[0], r)
        wdim_fb = _splat(pf[4], r)
        base_b = _splat(pi[1], r)
        xlo, xhi, wxl, wxh = _side(_splat(pf[0], r), _splat(pf[2], r), wdim_b,
                                   wdim_fb, off)
        ylo, yhi, wyl, wyh = _side(_splat(pf[1], r), _splat(pf[3], r), wdim_b,
                                   wdim_fb, off)

        for jy in range(15):
            rowlo = base_b + _splat(ylo, jy) * wdim_b
            rowhi = base_b + _splat(yhi, jy) * wdim_b
            idxbuf[jy, pl.ds(0, 16)] = rowlo + xlo
            idxbuf[jy, pl.ds(16, 16)] = rowlo + xhi
            idxbuf[jy, pl.ds(32, 16)] = rowhi + xlo
            idxbuf[jy, pl.ds(48, 16)] = rowhi + xhi
            if jy < 14:
                wyl_b = _splat(wyl, jy)
                wyh_b = _splat(wyh, jy)
                wbuf[4 * jy + 0] = wyl_b * wxl
                wbuf[4 * jy + 1] = wyl_b * wxh
                wbuf[4 * jy + 2] = wyh_b * wxl
                wbuf[4 * jy + 3] = wyh_b * wxh

        def fire(jy, buf, s):
            pltpu.async_copy(table.at[idxbuf.at[jy]], buf, s)

        def compute_row(jy, byrow, buf, first):
            wc = (wbuf[4 * jy + 0], wbuf[4 * jy + 1],
                  wbuf[4 * jy + 2], wbuf[4 * jy + 3])
            for bx in range(7):
                ws = []
                rows = []
                for c in range(4):
                    for p in (2 * bx, 2 * bx + 1):
                        ws.append(_splat(wc[c], p))
                        rows.append(c * 16 + p)
                row0 = byrow * 7 + bx

                def lds(p):
                    return [[buf[rr, pl.ds((2 * p + h) * 16, 16)]
                             for rr in rows] for h in (0, 1)]

                cur = lds(0)
                for p in range(_NCH // 2):
                    nxt = lds(p + 1) if p + 1 < _NCH // 2 else None
                    for h in (0, 1):
                        L = cur[h]
                        t0 = ws[0] * L[0] + ws[1] * L[1]
                        t1 = ws[2] * L[2] + ws[3] * L[3]
                        t2 = ws[4] * L[4] + ws[5] * L[5]
                        t3 = ws[6] * L[6] + ws[7] * L[7]
                        contrib = (t0 + t1) + (t2 + t3)
                        sl = pl.ds((2 * p + h) * 16, 16)
                        if first:
                            acc[row0, sl] = contrib
                        else:
                            plsc.addupdate(acc.at[row0, sl], contrib)
                    cur = nxt

        fire(0, rbuf_a, sem_a)

        def jj_body(jj, carry2):
            jy = 2 * jj
            fire(jy + 1, rbuf_b, sem_b)
            pltpu.make_async_copy(table.at[idxbuf.at[jy]], rbuf_a, sem_a).wait()
            compute_row(jy, jj, rbuf_a, True)
            fire(jy + 2, rbuf_a, sem_a)
            pltpu.make_async_copy(table.at[idxbuf.at[jy]], rbuf_b, sem_b).wait()
            compute_row(jy + 1, jj, rbuf_b, False)
            return carry2
        lax.fori_loop(0, 7, jj_body, 0)
        pltpu.make_async_copy(table.at[idxbuf.at[0]], rbuf_a, sem_a).wait()
        pltpu.sync_copy(acc, out.at[wid * 16 + r])
        return carry
    lax.fori_loop(0, 16, roi_body, 0)


def _tc_pack_level(feat, table, start_row, yb):
    B, C, S, _ = feat.shape
    rows_per_step = yb * S
    base = start_row // rows_per_step

    def body(f_ref, *t_refs):
        x = f_ref[0].reshape(C, yb * S)
        t_refs[-1][...] = jnp.transpose(x, (1, 0))

    in_specs = [pl.BlockSpec((1, C, yb, S), lambda b, g: (b, 0, g, 0))]
    args = (feat,)
    aliases = {}
    if table is not None:
        in_specs.append(pl.BlockSpec(memory_space=pl.ANY))
        args = (feat, table)
        aliases = {1: 0}
    return pl.pallas_call(
        body,
        grid=(B, S // yb),
        in_specs=in_specs,
        out_specs=pl.BlockSpec((rows_per_step, C),
                               lambda b, g, _S=S, _yb=yb, _base=base:
                               (_base + b * (_S // _yb) + g, 0)),
        out_shape=jax.ShapeDtypeStruct((174080, _C), F32),
        input_output_aliases=aliases,
    )(*args)


@jax.jit
def _roipool(table, boxes):
    mesh = plsc.VectorSubcoreMesh(core_axis_name="c", subcore_axis_name="s")
    fn = pl.kernel(
        _sc_body, mesh=mesh,
        out_type=jax.ShapeDtypeStruct((_NROI, 49, _C), F32),
        scratch_types=[
            pltpu.VMEM((4, 16), F32),
            pltpu.VMEM((5, 16), F32),
            pltpu.VMEM((2, 16), I32),
            pltpu.VMEM((15, 64), I32),
            pltpu.VMEM((56, 16), F32),
            pltpu.VMEM((64, _C), F32),
            pltpu.VMEM((64, _C), F32),
            pltpu.VMEM((49, _C), F32),
            pltpu.SemaphoreType.DMA,
            pltpu.SemaphoreType.DMA,
        ],
    )
    return fn(table, boxes)


def kernel(features_0, features_1, features_2, features_3, boxes_0, boxes_1):
    feats = (features_0, features_1, features_2, features_3)
    table = None
    for f, st, yb in zip(feats, _STARTS, (8, 16, 32, 32)):
        table = _tc_pack_level(f, table, st, yb)
    boxes = (jnp.concatenate([boxes_0, boxes_1], 0)
             .reshape(32, 16, 4).transpose(0, 2, 1))
    out = _roipool(table, boxes)
    return out.transpose(0, 2, 1).reshape(_NROI, _C, 7, 7)
